# baseline (speedup 1.0000x reference)
import functools

import jax
import jax.numpy as jnp
from jax import lax
from jax.experimental import pallas as pl
from jax.experimental.pallas import tpu as pltpu
from jax.experimental.pallas import tpu_sc as plsc

NC = 2
NS = 16
NW = NC * NS
LANES = 16
CHUNK = 128
NBLK = 3


def _zeros16():
    return jnp.zeros((LANES,), jnp.float32)


def _deg_body(dst_hbm, ew_hbm, out_hbm, deg_sh, dstbuf, ewbuf, zb):
    cid = lax.axis_index("c")
    sid = lax.axis_index("s")
    tg = sid * NC + cid
    npad = deg_sh.shape[0]
    per = npad // NS

    def zlane(i, _):
        zb[pl.ds(i * LANES, LANES)] = _zeros16()
        return 0

    lax.fori_loop(0, per // LANES, zlane, 0)
    pltpu.sync_copy(zb, deg_sh.at[pl.ds(sid * per, per)])
    pltpu.sync_copy(dst_hbm.at[tg], dstbuf)
    pltpu.sync_copy(ew_hbm.at[tg], ewbuf)
    plsc.subcore_barrier()

    def chunk(c, _):
        pltpu.sync_copy(ewbuf.at[c], deg_sh.at[dstbuf.at[c]], add=True)
        return 0

    lax.fori_loop(0, dstbuf.shape[0], chunk, 0)
    plsc.subcore_barrier()
    pltpu.sync_copy(deg_sh.at[pl.ds(sid * per, per)],
                    out_hbm.at[cid, pl.ds(sid * per, per)])


def _norm_body(src_hbm, dst_hbm, ew_hbm, dis_hbm, out_hbm,
               dis_v, sbuf, dbuf, ebuf, nbuf):
    cid = lax.axis_index("c")
    sid = lax.axis_index("s")
    tg = sid * NC + cid
    ept = src_hbm.shape[1]
    nb = sbuf.shape[0]
    pltpu.sync_copy(dis_hbm, dis_v)

    def blk(bi, _):
        off = pl.ds(bi * nb, nb)
        pltpu.sync_copy(src_hbm.at[tg, off], sbuf)
        pltpu.sync_copy(dst_hbm.at[tg, off], dbuf)
        pltpu.sync_copy(ew_hbm.at[tg, off], ebuf)

        def grp(g, _):
            s16 = sbuf[pl.ds(g * LANES, LANES)]
            d16 = dbuf[pl.ds(g * LANES, LANES)]
            e16 = ebuf[pl.ds(g * LANES, LANES)]
            nbuf[pl.ds(g * LANES, LANES)] = (
                e16 * plsc.load_gather(dis_v, [s16])
                * plsc.load_gather(dis_v, [d16]))
            return 0

        lax.fori_loop(0, nb // LANES, grp, 0)
        pltpu.sync_copy(nbuf, out_hbm.at[tg, off])
        return 0

    lax.fori_loop(0, ept // nb, blk, 0)


def _msg_body(x_hbm, src_hbm, nrm_hbm, dst_hbm, out_hbm,
              acc_sh, rows0, rows1, srcq, nrmq, dstq,
              gsem0, gsem1, ssem0, ssem1):
    cid = lax.axis_index("c")
    sid = lax.axis_index("s")
    tg = sid * NC + cid
    n = acc_sh.shape[0]
    rpt = n // NS
    nw = srcq.shape[0]
    npc = nw // CHUNK

    rows = (rows0, rows1)
    gsem = (gsem0, gsem1)
    ssem = (ssem0, ssem1)

    def zrow(e, _):
        for j in range(8):
            rows0[e, pl.ds(j * LANES, LANES)] = _zeros16()
        return 0

    lax.fori_loop(0, CHUNK, zrow, 0)
    for q in range(rpt // CHUNK):
        pltpu.sync_copy(rows0, acc_sh.at[pl.ds(sid * rpt + q * CHUNK, CHUNK)])
    plsc.subcore_barrier()

    def issue_gather(p, b):
        pltpu.async_copy(x_hbm.at[srcq.at[pl.ds(p * CHUNK, CHUNK)]],
                         rows[b], gsem[b])

    def wait_gather(p, b):
        pltpu.make_async_copy(x_hbm.at[srcq.at[pl.ds(p * CHUNK, CHUNK)]],
                              rows[b], gsem[b]).wait()

    def issue_scat(p, b):
        pltpu.async_copy(rows[b], acc_sh.at[dstq.at[p]], ssem[b], add=True)

    def wait_scat(p, b):
        pltpu.make_async_copy(rows[b], acc_sh.at[dstq.at[p]],
                              ssem[b]).wait()

    def proc(p, k):
        b = k

        @pl.when(p + 1 < npc)
        def _():
            issue_gather(p + 1, 1 - b)

        @plsc.parallel_loop(0, CHUNK, 1, unroll=2)
        def _(e):
            nb16 = plsc.load_gather(
                nrmq, [jnp.full((LANES,), p * CHUNK + e, jnp.int32)])
            for j in range(8):
                rows[b][e, pl.ds(j * LANES, LANES)] = (
                    rows[b][e, pl.ds(j * LANES, LANES)] * nb16)

        @pl.when(p + 1 < npc)
        def _():
            wait_gather(p + 1, 1 - b)

        issue_scat(p, b)
        wait_scat(p, b)

    for t in range(NBLK):
        off = pl.ds(t * nw, nw)
        pltpu.sync_copy(src_hbm.at[tg, off], srcq)
        pltpu.sync_copy(nrm_hbm.at[tg, off], nrmq)
        pltpu.sync_copy(dst_hbm.at[tg, t], dstq)
        issue_gather(0, 0)
        wait_gather(0, 0)

        def pair(j, _):
            proc(2 * j, 0)
            proc(2 * j + 1, 1)
            return 0

        lax.fori_loop(0, npc // 2, pair, 0)

    plsc.subcore_barrier()
    for q in range(rpt // CHUNK):
        r0 = sid * rpt + q * CHUNK
        pltpu.sync_copy(acc_sh.at[pl.ds(r0, CHUNK)],
                        out_hbm.at[cid, pl.ds(r0, CHUNK)])


def _dis_body(degp_ref, dis_ref):
    d = degp_ref[0] + degp_ref[1]
    dis_ref[...] = jnp.where(d > 0, lax.rsqrt(jnp.where(d > 0, d, 1.0)), 0.0)


def _out_body(m_ref, w_ref, b_ref, o_ref):
    a = m_ref[0] + m_ref[1]
    o_ref[...] = jnp.maximum(
        jnp.dot(a, w_ref[...], preferred_element_type=jnp.float32)
        + b_ref[...], 0.0)


def kernel(x, edge_index, edge_weight, W, b):
    x = x.astype(jnp.float32)
    N, D = x.shape
    E = edge_index.shape[1]
    src = edge_index[0].astype(jnp.int32)
    dst = edge_index[1].astype(jnp.int32)
    ew = edge_weight.astype(jnp.float32)

    loop_idx = jnp.arange(N, dtype=jnp.int32)
    e_all = E + N
    step = 2 * NBLK * CHUNK
    ept = -(-(-(-e_all // NW)) // step) * step
    padn = ept * NW - e_all
    nch = ept // CHUNK
    nw = ept // NBLK

    src_all = jnp.concatenate([src, loop_idx, jnp.zeros((padn,), jnp.int32)])
    dst_all = jnp.concatenate([dst, loop_idx, jnp.zeros((padn,), jnp.int32)])
    ew_all = jnp.concatenate(
        [ew, jnp.ones((N,), jnp.float32), jnp.zeros((padn,), jnp.float32)])
    src_f = src_all.reshape(NW, ept)
    dst_f = dst_all.reshape(NW, ept)
    ew_f = ew_all.reshape(NW, ept)
    dst_a = dst_all.reshape(NW, nch, CHUNK)
    ew_a = ew_all.reshape(NW, nch, CHUNK)
    dst_4 = dst_all.reshape(NW, NBLK, nch // NBLK, CHUNK)

    npad = -(-N // 256) * 256

    mesh = plsc.VectorSubcoreMesh(core_axis_name="c", subcore_axis_name="s",
                                  num_cores=NC, num_subcores=NS)

    deg_call = pl.kernel(
        _deg_body,
        out_type=jax.ShapeDtypeStruct((NC, npad), jnp.float32),
        mesh=mesh,
        scratch_types=[
            pltpu.VMEM_SHARED((npad,), jnp.float32),
            pltpu.VMEM((nch, CHUNK), jnp.int32),
            pltpu.VMEM((nch, CHUNK), jnp.float32),
            pltpu.VMEM((npad // NS,), jnp.float32),
        ],
    )
    degp = deg_call(dst_a, ew_a)

    dis = pl.pallas_call(
        _dis_body,
        out_shape=jax.ShapeDtypeStruct((npad // 128, 128), jnp.float32),
    )(degp.reshape(NC, npad // 128, 128)).reshape(npad)

    norm_call = pl.kernel(
        _norm_body,
        out_type=jax.ShapeDtypeStruct((NW, ept), jnp.float32),
        mesh=mesh,
        scratch_types=[
            pltpu.VMEM((npad,), jnp.float32),
            pltpu.VMEM((nw,), jnp.int32),
            pltpu.VMEM((nw,), jnp.int32),
            pltpu.VMEM((nw,), jnp.float32),
            pltpu.VMEM((nw,), jnp.float32),
        ],
        compiler_params=pltpu.CompilerParams(needs_layout_passes=False),
    )
    nrm_f = norm_call(src_f, dst_f, ew_f, dis)

    msg_call = pl.kernel(
        _msg_body,
        out_type=jax.ShapeDtypeStruct((NC, npad, D), jnp.float32),
        mesh=mesh,
        scratch_types=[
            pltpu.VMEM_SHARED((npad, D), jnp.float32),
            pltpu.VMEM((CHUNK, D), jnp.float32),
            pltpu.VMEM((CHUNK, D), jnp.float32),
            pltpu.VMEM((nw,), jnp.int32),
            pltpu.VMEM((nw,), jnp.float32),
            pltpu.VMEM((nch // NBLK, CHUNK), jnp.int32),
        ] + [pltpu.SemaphoreType.DMA] * 4,
        compiler_params=pltpu.CompilerParams(needs_layout_passes=False),
    )
    M = msg_call(x, src_f, nrm_f, dst_4)

    BM = 1000
    out = pl.pallas_call(
        _out_body,
        grid=(N // BM,),
        in_specs=[
            pl.BlockSpec((NC, BM, D), lambda i: (0, i, 0)),
            pl.BlockSpec((D, D), lambda i: (0, 0)),
            pl.BlockSpec((1, D), lambda i: (0, 0)),
        ],
        out_specs=pl.BlockSpec((BM, D), lambda i: (i, 0)),
        out_shape=jax.ShapeDtypeStruct((N, D), jnp.float32),
    )(M, W, b.reshape(1, D))
    return out


# speedup vs baseline: 1.8281x; 1.8281x over previous
import functools

import jax
import jax.numpy as jnp
from jax import lax
from jax.experimental import pallas as pl
from jax.experimental.pallas import tpu as pltpu
from jax.experimental.pallas import tpu_sc as plsc

NC = 2
NS = 16
NW = NC * NS
LANES = 16
CHUNK = 128


def _zeros16():
    return jnp.zeros((LANES,), jnp.float32)


def _deg_body(dst_hbm, ew_hbm, out_hbm, deg_sh, dstbuf, ewbuf, zb):
    cid = lax.axis_index("c")
    sid = lax.axis_index("s")
    tg = cid * NS + sid
    npad = deg_sh.shape[0]
    per = npad // NS

    def zlane(i, _):
        zb[pl.ds(i * LANES, LANES)] = _zeros16()
        return 0

    lax.fori_loop(0, per // LANES, zlane, 0)
    pltpu.sync_copy(zb, deg_sh.at[pl.ds(sid * per, per)])
    pltpu.sync_copy(dst_hbm.at[tg], dstbuf)
    pltpu.sync_copy(ew_hbm.at[tg], ewbuf)
    plsc.subcore_barrier()

    def chunk(c, _):
        pltpu.sync_copy(ewbuf.at[c], deg_sh.at[dstbuf.at[c]], add=True)
        return 0

    lax.fori_loop(0, dstbuf.shape[0], chunk, 0)
    plsc.subcore_barrier()
    pltpu.sync_copy(deg_sh.at[pl.ds(sid * per, per)],
                    out_hbm.at[cid, pl.ds(sid * per, per)])


def _msg_body(x_hbm, src_hbm, dst_hbm, ew_hbm, dis_hbm, out_hbm,
              acc_sh, dis_v, srcbuf, dstbuf, ewbuf, rows_v, normbuf):
    cid = lax.axis_index("c")
    sid = lax.axis_index("s")
    tg = cid * NS + sid
    n = acc_sh.shape[0]
    rpt = n // NS
    nw = srcbuf.shape[0]
    nbs = nw // CHUNK
    nblk = dstbuf.shape[0] // nbs

    def zrow(e, _):
        for j in range(8):
            rows_v[e, pl.ds(j * LANES, LANES)] = _zeros16()
        return 0

    lax.fori_loop(0, CHUNK, zrow, 0)
    for q in range(rpt // CHUNK):
        pltpu.sync_copy(rows_v, acc_sh.at[pl.ds(sid * rpt + q * CHUNK, CHUNK)])
    pltpu.sync_copy(dis_hbm, dis_v)
    pltpu.sync_copy(dst_hbm.at[tg], dstbuf)
    plsc.subcore_barrier()

    def block(bi, _):
        pltpu.sync_copy(src_hbm.at[tg, pl.ds(bi * nw, nw)], srcbuf)
        pltpu.sync_copy(ew_hbm.at[tg, pl.ds(bi * nw, nw)], ewbuf)

        def chunk(c, _):
            cg = bi * nbs + c
            pltpu.sync_copy(x_hbm.at[srcbuf.at[pl.ds(c * CHUNK, CHUNK)]],
                            rows_v)

            def grp(g, _):
                s16 = srcbuf[pl.ds(c * CHUNK + g * LANES, LANES)]
                d16 = dstbuf[cg, pl.ds(g * LANES, LANES)]
                e16 = ewbuf[pl.ds(c * CHUNK + g * LANES, LANES)]
                nv = e16 * plsc.load_gather(dis_v, [s16]) \
                         * plsc.load_gather(dis_v, [d16])
                normbuf[pl.ds(g * LANES, LANES)] = nv
                return 0

            lax.fori_loop(0, CHUNK // LANES, grp, 0)

            def edge(e, _):
                nb = plsc.load_gather(
                    normbuf, [jnp.full((LANES, ), e, jnp.int32)])
                for j in range(8):
                    rows_v[e, pl.ds(j * LANES, LANES)] = (
                        rows_v[e, pl.ds(j * LANES, LANES)] * nb)
                return 0

            lax.fori_loop(0, CHUNK, edge, 0)
            pltpu.sync_copy(rows_v, acc_sh.at[dstbuf.at[cg]], add=True)
            return 0

        lax.fori_loop(0, nbs, chunk, 0)
        return 0

    lax.fori_loop(0, nblk, block, 0)
    plsc.subcore_barrier()
    for q in range(rpt // CHUNK):
        r0 = sid * rpt + q * CHUNK
        pltpu.sync_copy(acc_sh.at[pl.ds(r0, CHUNK)],
                        out_hbm.at[cid, pl.ds(r0, CHUNK)])


def _dis_body(degp_ref, dis_ref):
    d = degp_ref[0] + degp_ref[1]
    dis_ref[...] = jnp.where(d > 0, lax.rsqrt(jnp.where(d > 0, d, 1.0)), 0.0)


def _out_body(m_ref, w_ref, b_ref, o_ref):
    a = m_ref[0] + m_ref[1]
    o_ref[...] = jnp.maximum(
        jnp.dot(a, w_ref[...], preferred_element_type=jnp.float32)
        + b_ref[...], 0.0)


def kernel(x, edge_index, edge_weight, W, b):
    x = x.astype(jnp.float32)
    N, D = x.shape
    E = edge_index.shape[1]
    src = edge_index[0].astype(jnp.int32)
    dst = edge_index[1].astype(jnp.int32)
    ew = edge_weight.astype(jnp.float32)

    loop_idx = jnp.arange(N, dtype=jnp.int32)
    e_all = E + N
    step = 3 * CHUNK
    ept = -(-(-(-e_all // NW)) // step) * step
    padn = ept * NW - e_all
    nch = ept // CHUNK

    src_all = jnp.concatenate([src, loop_idx, jnp.zeros((padn,), jnp.int32)])
    dst_all = jnp.concatenate([dst, loop_idx, jnp.zeros((padn,), jnp.int32)])
    ew_all = jnp.concatenate(
        [ew, jnp.ones((N,), jnp.float32), jnp.zeros((padn,), jnp.float32)])
    src_f = src_all.reshape(NW, ept)
    ew_f = ew_all.reshape(NW, ept)
    dst_a = dst_all.reshape(NW, nch, CHUNK)
    ew_a = ew_all.reshape(NW, nch, CHUNK)

    npad = -(-N // 256) * 256

    mesh = plsc.VectorSubcoreMesh(core_axis_name="c", subcore_axis_name="s",
                                  num_cores=NC, num_subcores=NS)

    deg_call = pl.kernel(
        _deg_body,
        out_type=jax.ShapeDtypeStruct((NC, npad), jnp.float32),
        mesh=mesh,
        scratch_types=[
            pltpu.VMEM_SHARED((npad,), jnp.float32),
            pltpu.VMEM((nch, CHUNK), jnp.int32),
            pltpu.VMEM((nch, CHUNK), jnp.float32),
            pltpu.VMEM((npad // NS,), jnp.float32),
        ],
    )
    degp = deg_call(dst_a, ew_a)

    dis = pl.pallas_call(
        _dis_body,
        out_shape=jax.ShapeDtypeStruct((npad // 128, 128), jnp.float32),
    )(degp.reshape(NC, npad // 128, 128)).reshape(npad)

    nw = (nch // 3) * CHUNK
    msg_call = pl.kernel(
        _msg_body,
        out_type=jax.ShapeDtypeStruct((NC, npad, D), jnp.float32),
        mesh=mesh,
        scratch_types=[
            pltpu.VMEM_SHARED((npad, D), jnp.float32),
            pltpu.VMEM((npad,), jnp.float32),
            pltpu.VMEM((nw,), jnp.int32),
            pltpu.VMEM((nch, CHUNK), jnp.int32),
            pltpu.VMEM((nw,), jnp.float32),
            pltpu.VMEM((CHUNK, D), jnp.float32),
            pltpu.VMEM((CHUNK,), jnp.float32),
        ],
        compiler_params=pltpu.CompilerParams(needs_layout_passes=False),
    )
    M = msg_call(x, src_f, dst_a, ew_f, dis)

    BM = 1000
    out = pl.pallas_call(
        _out_body,
        grid=(N // BM,),
        in_specs=[
            pl.BlockSpec((NC, BM, D), lambda i: (0, i, 0)),
            pl.BlockSpec((D, D), lambda i: (0, 0)),
            pl.BlockSpec((1, D), lambda i: (0, 0)),
        ],
        out_specs=pl.BlockSpec((BM, D), lambda i: (i, 0)),
        out_shape=jax.ShapeDtypeStruct((N, D), jnp.float32),
    )(M, W, b.reshape(1, D))
    return out
